# baseline (speedup 1.0000x reference)
import functools

import jax
import jax.numpy as jnp
from jax import lax
from jax.experimental import pallas as pl
from jax.experimental.pallas import tpu as pltpu
from jax.experimental.pallas import tpu_sc as plsc

B = 16384
L = 50
H = 32
L2 = 2 * L
NUM_CORES = 2
NUM_SUBCORES = 16
NW = NUM_CORES * NUM_SUBCORES
BPW = B // NW
CB = 32
NCHUNK = BPW // CB
NPAIR = NCHUNK // 2
IPC = CB * L2
GSZ = 128
INV_L = 1.0 / L

_SPLITS = []
_off = 0
while _off < IPC:
    _n = min(GSZ, IPC - _off)
    _SPLITS.append((_off, _n))
    _off += _n


def _fire(table_hbm, idx_v, rows_v, sem):
    for off, n in _SPLITS:
        pltpu.async_copy(
            table_hbm.at[idx_v.at[pl.ds(off, n)]],
            rows_v.at[pl.ds(off, n)],
            sem,
        )


def _drain(table_hbm, idx_v, rows_v, sem):
    for off, n in _SPLITS:
        pltpu.make_async_copy(
            table_hbm.at[idx_v.at[pl.ds(off, n)]],
            rows_v.at[pl.ds(off, n)],
            sem,
        ).wait()


def _accum_store(rows_v, out_v, out_hbm, row0):
    def row_body(r, carry):
        off = r * L2
        acc0 = jnp.zeros((16,), jnp.float32)
        acc1 = jnp.zeros((16,), jnp.float32)
        for j in range(L):
            acc0 = acc0 + rows_v[off + 2 * j, :]
            acc1 = acc1 + rows_v[off + 2 * j + 1, :]
        out_v[r, pl.ds(0, 16)] = acc0 * INV_L
        out_v[r, pl.ds(16, 16)] = acc1 * INV_L
        return carry

    lax.fori_loop(0, CB, row_body, 0)
    pltpu.sync_copy(out_v, out_hbm.at[pl.ds(row0, CB)])


def _embed_body(ids_hbm, table_hbm, out_hbm,
                idx0, idx1, rows0, rows1, out_v, sem0, sem1):
    c = lax.axis_index("c")
    s = lax.axis_index("s")
    wid = s * NUM_CORES + c
    base = wid * BPW

    pltpu.sync_copy(ids_hbm.at[pl.ds(base * L2, IPC)], idx0)
    _fire(table_hbm, idx0, rows0, sem0)

    def pair_body(i, carry):
        row_a = base + (2 * i) * CB
        row_b = row_a + CB
        pltpu.sync_copy(ids_hbm.at[pl.ds(row_b * L2, IPC)], idx1)
        _fire(table_hbm, idx1, rows1, sem1)
        _drain(table_hbm, idx0, rows0, sem0)
        _accum_store(rows0, out_v, out_hbm, row_a)

        @pl.when(i < NPAIR - 1)
        def _():
            row_c = row_b + CB
            pltpu.sync_copy(ids_hbm.at[pl.ds(row_c * L2, IPC)], idx0)
            _fire(table_hbm, idx0, rows0, sem0)

        _drain(table_hbm, idx1, rows1, sem1)
        _accum_store(rows1, out_v, out_hbm, row_b)
        return carry

    lax.fori_loop(0, NPAIR, pair_body, 0)


@jax.jit
def _embed(ids2_flat, table_half):
    mesh = plsc.VectorSubcoreMesh(
        core_axis_name="c",
        subcore_axis_name="s",
        num_cores=NUM_CORES,
        num_subcores=NUM_SUBCORES,
    )
    return pl.kernel(
        _embed_body,
        out_type=jax.ShapeDtypeStruct((B, H), jnp.float32),
        mesh=mesh,
        scratch_types=[
            pltpu.VMEM((IPC,), jnp.int32),
            pltpu.VMEM((IPC,), jnp.int32),
            pltpu.VMEM((IPC, 16), jnp.float32),
            pltpu.VMEM((IPC, 16), jnp.float32),
            pltpu.VMEM((CB, H), jnp.float32),
            pltpu.SemaphoreType.DMA,
            pltpu.SemaphoreType.DMA,
        ],
        compiler_params=pltpu.CompilerParams(
            use_tc_tiling_on_sc=False, needs_layout_passes=False
        ),
    )(ids2_flat, table_half)


def kernel(instruction_ids, embed_weight):
    ids = instruction_ids.astype(jnp.int32)
    ids2 = jnp.stack((2 * ids, 2 * ids + 1), axis=-1).reshape(-1)
    table_half = embed_weight.reshape(2 * 1000000, 16)
    return _embed(ids2, table_half)


# speedup vs baseline: 2.7931x; 2.7931x over previous
import functools

import jax
import jax.numpy as jnp
from jax import lax
from jax.experimental import pallas as pl
from jax.experimental.pallas import tpu as pltpu
from jax.experimental.pallas import tpu_sc as plsc

B = 16384
L = 50
H = 32
NUM_CORES = 2
NUM_SUBCORES = 16
NW = NUM_CORES * NUM_SUBCORES
BPW = B // NW
CB = 32
NCHUNK = BPW // CB
NPAIR = NCHUNK // 2
IPC = CB * L
GSZ = 128
INV_L = 1.0 / L

_SPLITS = []
_off = 0
while _off < IPC:
    _n = min(GSZ, IPC - _off)
    _SPLITS.append((_off, _n))
    _off += _n


def _fire(table_hbm, idx_v, rows_v, sem):
    for off, n in _SPLITS:
        pltpu.async_copy(
            table_hbm.at[idx_v.at[pl.ds(off, n)]],
            rows_v.at[pl.ds(off, n)],
            sem,
        )


def _drain(table_hbm, idx_v, rows_v, sem):
    for off, n in _SPLITS:
        pltpu.make_async_copy(
            table_hbm.at[idx_v.at[pl.ds(off, n)]],
            rows_v.at[pl.ds(off, n)],
            sem,
        ).wait()


def _stage_ids(ids_hbm, col0, idx_v, sem):
    for l in range(L):
        pltpu.async_copy(
            ids_hbm.at[l, pl.ds(col0, CB)],
            idx_v.at[pl.ds(l * CB, CB)],
            sem,
        )
    for l in range(L):
        pltpu.make_async_copy(
            ids_hbm.at[l, pl.ds(col0, CB)],
            idx_v.at[pl.ds(l * CB, CB)],
            sem,
        ).wait()


def _accum_store(rows_v, out_v, out_hbm, row0):
    def row_body(b, carry):
        acc0 = jnp.zeros((16,), jnp.float32)
        acc1 = jnp.zeros((16,), jnp.float32)
        for l in range(L):
            acc0 = acc0 + rows_v[l * CB + b, pl.ds(0, 16)]
            acc1 = acc1 + rows_v[l * CB + b, pl.ds(16, 16)]
        out_v[b, pl.ds(0, 16)] = acc0 * INV_L
        out_v[b, pl.ds(16, 16)] = acc1 * INV_L
        return carry

    lax.fori_loop(0, CB, row_body, 0)
    pltpu.sync_copy(out_v, out_hbm.at[pl.ds(row0, CB)])


def _embed_body(ids_hbm, table_hbm, out_hbm,
                idx0, idx1, rows0, rows1, out_v, sem0, sem1, semi0, semi1):
    c = lax.axis_index("c")
    s = lax.axis_index("s")
    wid = s * NUM_CORES + c
    base = wid * BPW

    _stage_ids(ids_hbm, base, idx0, semi0)
    _fire(table_hbm, idx0, rows0, sem0)

    def pair_body(i, carry):
        row_a = base + (2 * i) * CB
        row_b = row_a + CB
        _stage_ids(ids_hbm, row_b, idx1, semi1)
        _fire(table_hbm, idx1, rows1, sem1)
        _drain(table_hbm, idx0, rows0, sem0)
        _accum_store(rows0, out_v, out_hbm, row_a)

        @pl.when(i < NPAIR - 1)
        def _():
            row_c = row_b + CB
            _stage_ids(ids_hbm, row_c, idx0, semi0)
            _fire(table_hbm, idx0, rows0, sem0)

        _drain(table_hbm, idx1, rows1, sem1)
        _accum_store(rows1, out_v, out_hbm, row_b)
        return carry

    lax.fori_loop(0, NPAIR, pair_body, 0)


@jax.jit
def _embed(ids_t, table):
    mesh = plsc.VectorSubcoreMesh(
        core_axis_name="c",
        subcore_axis_name="s",
        num_cores=NUM_CORES,
        num_subcores=NUM_SUBCORES,
    )
    return pl.kernel(
        _embed_body,
        out_type=jax.ShapeDtypeStruct((B, H), jnp.float32),
        mesh=mesh,
        scratch_types=[
            pltpu.VMEM((IPC,), jnp.int32),
            pltpu.VMEM((IPC,), jnp.int32),
            pltpu.VMEM((IPC, H), jnp.float32),
            pltpu.VMEM((IPC, H), jnp.float32),
            pltpu.VMEM((CB, H), jnp.float32),
            pltpu.SemaphoreType.DMA,
            pltpu.SemaphoreType.DMA,
            pltpu.SemaphoreType.DMA,
            pltpu.SemaphoreType.DMA,
        ],
        compiler_params=pltpu.CompilerParams(use_tc_tiling_on_sc=False),
    )(ids_t, table)


def kernel(instruction_ids, embed_weight):
    ids_t = instruction_ids.astype(jnp.int32).T
    return _embed(ids_t, embed_weight)


# speedup vs baseline: 2.8393x; 1.0165x over previous
import functools

import jax
import jax.numpy as jnp
from jax import lax
from jax.experimental import pallas as pl
from jax.experimental.pallas import tpu as pltpu
from jax.experimental.pallas import tpu_sc as plsc

B = 16384
L = 50
H = 32
NUM_CORES = 2
NUM_SUBCORES = 16
NW = NUM_CORES * NUM_SUBCORES
BPW = B // NW
CB = 32
NCHUNK = BPW // CB
NPAIR = NCHUNK // 2
IPC = CB * L
GSZ = 128
INV_L = 1.0 / L

_SPLITS = []
_off = 0
while _off < IPC:
    _n = min(GSZ, IPC - _off)
    _SPLITS.append((_off, _n))
    _off += _n


def _fire(table_hbm, idx_v, rows_v, sem):
    for off, n in _SPLITS:
        pltpu.async_copy(
            table_hbm.at[idx_v.at[pl.ds(off, n)]],
            rows_v.at[pl.ds(off, n)],
            sem,
        )


def _drain(table_hbm, idx_v, rows_v, sem):
    for off, n in _SPLITS:
        pltpu.make_async_copy(
            table_hbm.at[idx_v.at[pl.ds(off, n)]],
            rows_v.at[pl.ds(off, n)],
            sem,
        ).wait()


def _stage_ids(ids_hbm, col0, idx_v, sem):
    for l in range(L):
        pltpu.async_copy(
            ids_hbm.at[pl.ds(l * B + col0, CB)],
            idx_v.at[pl.ds(l * CB, CB)],
            sem,
        )
    for l in range(L):
        pltpu.make_async_copy(
            ids_hbm.at[pl.ds(l * B + col0, CB)],
            idx_v.at[pl.ds(l * CB, CB)],
            sem,
        ).wait()


def _flatten_ids_body(ids_t_hbm, out_hbm, stg, sem):
    c = lax.axis_index("c")
    s = lax.axis_index("s")
    wid = s * NUM_CORES + c
    col0 = wid * (B // NW)
    pltpu.sync_copy(ids_t_hbm.at[pl.ds(0, L), pl.ds(col0, B // NW)], stg)
    for l in range(L):
        pltpu.async_copy(
            stg.at[l, :],
            out_hbm.at[pl.ds(l * B + col0, B // NW)],
            sem,
        )
    for l in range(L):
        pltpu.make_async_copy(
            stg.at[l, :],
            out_hbm.at[pl.ds(l * B + col0, B // NW)],
            sem,
        ).wait()


def _accum_store(rows_v, out_v, out_hbm, row0):
    def row_body(b, carry):
        acc0 = jnp.zeros((16,), jnp.float32)
        acc1 = jnp.zeros((16,), jnp.float32)
        for l in range(L):
            acc0 = acc0 + rows_v[l * CB + b, pl.ds(0, 16)]
            acc1 = acc1 + rows_v[l * CB + b, pl.ds(16, 16)]
        out_v[b, pl.ds(0, 16)] = acc0 * INV_L
        out_v[b, pl.ds(16, 16)] = acc1 * INV_L
        return carry

    lax.fori_loop(0, CB, row_body, 0)
    pltpu.sync_copy(out_v, out_hbm.at[pl.ds(row0, CB)])


def _embed_body(ids_hbm, table_hbm, out_hbm,
                idx0, idx1, rows0, rows1, out_v, sem0, sem1, semi0, semi1):
    c = lax.axis_index("c")
    s = lax.axis_index("s")
    wid = s * NUM_CORES + c
    base = wid * BPW

    _stage_ids(ids_hbm, base, idx0, semi0)
    _fire(table_hbm, idx0, rows0, sem0)

    def pair_body(i, carry):
        row_a = base + (2 * i) * CB
        row_b = row_a + CB
        _stage_ids(ids_hbm, row_b, idx1, semi1)
        _fire(table_hbm, idx1, rows1, sem1)
        _drain(table_hbm, idx0, rows0, sem0)
        _accum_store(rows0, out_v, out_hbm, row_a)

        @pl.when(i < NPAIR - 1)
        def _():
            row_c = row_b + CB
            _stage_ids(ids_hbm, row_c, idx0, semi0)
            _fire(table_hbm, idx0, rows0, sem0)

        _drain(table_hbm, idx1, rows1, sem1)
        _accum_store(rows1, out_v, out_hbm, row_b)
        return carry

    lax.fori_loop(0, NPAIR, pair_body, 0)


@jax.jit
def _embed(ids_t, table):
    mesh = plsc.VectorSubcoreMesh(
        core_axis_name="c",
        subcore_axis_name="s",
        num_cores=NUM_CORES,
        num_subcores=NUM_SUBCORES,
    )
    ids_flat = pl.kernel(
        _flatten_ids_body,
        out_type=jax.ShapeDtypeStruct((L * B,), jnp.int32),
        mesh=mesh,
        scratch_types=[
            pltpu.VMEM((L, B // NW), jnp.int32),
            pltpu.SemaphoreType.DMA,
        ],
        compiler_params=pltpu.CompilerParams(use_tc_tiling_on_sc=True),
    )(ids_t)
    return pl.kernel(
        _embed_body,
        out_type=jax.ShapeDtypeStruct((B, H), jnp.float32),
        mesh=mesh,
        scratch_types=[
            pltpu.VMEM((IPC,), jnp.int32),
            pltpu.VMEM((IPC,), jnp.int32),
            pltpu.VMEM((IPC, H), jnp.float32),
            pltpu.VMEM((IPC, H), jnp.float32),
            pltpu.VMEM((CB, H), jnp.float32),
            pltpu.SemaphoreType.DMA,
            pltpu.SemaphoreType.DMA,
            pltpu.SemaphoreType.DMA,
            pltpu.SemaphoreType.DMA,
        ],
        compiler_params=pltpu.CompilerParams(use_tc_tiling_on_sc=False),
    )(ids_flat, table)


def kernel(instruction_ids, embed_weight):
    ids_t = instruction_ids.astype(jnp.int32).T
    return _embed(ids_t, embed_weight)


# speedup vs baseline: 2.8428x; 1.0013x over previous
import functools

import jax
import jax.numpy as jnp
from jax import lax
from jax.experimental import pallas as pl
from jax.experimental.pallas import tpu as pltpu
from jax.experimental.pallas import tpu_sc as plsc

B = 16384
L = 50
H = 32
NUM_CORES = 2
NUM_SUBCORES = 16
NW = NUM_CORES * NUM_SUBCORES
BPW = B // NW
CB = 32
NCHUNK = BPW // CB
NPAIR = NCHUNK // 2
IPC = CB * L
GSZ = 128
INV_L = 1.0 / L

_SPLITS = []
_off = 0
while _off < IPC:
    _n = min(GSZ, IPC - _off)
    _SPLITS.append((_off, _n))
    _off += _n


def _fire(table_hbm, idx_v, rows_v, sem):
    for off, n in _SPLITS:
        pltpu.async_copy(
            table_hbm.at[idx_v.at[pl.ds(off, n)]],
            rows_v.at[pl.ds(off, n)],
            sem,
        )


def _drain(table_hbm, idx_v, rows_v, sem):
    for off, n in _SPLITS:
        pltpu.make_async_copy(
            table_hbm.at[idx_v.at[pl.ds(off, n)]],
            rows_v.at[pl.ds(off, n)],
            sem,
        ).wait()


def _stage_ids(ids_hbm, col0, idx_v, sem):
    for l in range(L):
        pltpu.async_copy(
            ids_hbm.at[pl.ds(l * B + col0, CB)],
            idx_v.at[pl.ds(l * CB, CB)],
            sem,
        )
    for l in range(L):
        pltpu.make_async_copy(
            ids_hbm.at[pl.ds(l * B + col0, CB)],
            idx_v.at[pl.ds(l * CB, CB)],
            sem,
        ).wait()


def _flatten_ids_body(ids_t_hbm, out_hbm, stg, sem):
    c = lax.axis_index("c")
    s = lax.axis_index("s")
    wid = s * NUM_CORES + c
    col0 = wid * (B // NW)
    pltpu.sync_copy(ids_t_hbm.at[pl.ds(0, L), pl.ds(col0, B // NW)], stg)
    for l in range(L):
        pltpu.async_copy(
            stg.at[l, :],
            out_hbm.at[pl.ds(l * B + col0, B // NW)],
            sem,
        )
    for l in range(L):
        pltpu.make_async_copy(
            stg.at[l, :],
            out_hbm.at[pl.ds(l * B + col0, B // NW)],
            sem,
        ).wait()


def _accum_store(rows_v, out_v, out_hbm, row0):
    def row_body(b, carry):
        acc0 = jnp.zeros((16,), jnp.float32)
        acc1 = jnp.zeros((16,), jnp.float32)
        for l in range(L):
            acc0 = acc0 + rows_v[l * CB + b, pl.ds(0, 16)]
            acc1 = acc1 + rows_v[l * CB + b, pl.ds(16, 16)]
        out_v[b, pl.ds(0, 16)] = acc0 * INV_L
        out_v[b, pl.ds(16, 16)] = acc1 * INV_L
        return carry

    lax.fori_loop(0, CB, row_body, 0)
    pltpu.sync_copy(out_v, out_hbm.at[pl.ds(row0, CB)])


def _embed_body(ids_hbm, table_hbm, out_hbm,
                idx0, idx1, rows0, rows1, out_v, sem0, sem1, semi0, semi1):
    c = lax.axis_index("c")
    s = lax.axis_index("s")
    wid = s * NUM_CORES + c
    base = wid * BPW

    _stage_ids(ids_hbm, base, idx0, semi0)
    _fire(table_hbm, idx0, rows0, sem0)

    def pair_body(i, carry):
        row_a = base + (2 * i) * CB
        row_b = row_a + CB
        _stage_ids(ids_hbm, row_b, idx1, semi1)
        _fire(table_hbm, idx1, rows1, sem1)
        _drain(table_hbm, idx0, rows0, sem0)
        _accum_store(rows0, out_v, out_hbm, row_a)

        @pl.when(i < NPAIR - 1)
        def _():
            row_c = row_b + CB
            _stage_ids(ids_hbm, row_c, idx0, semi0)
            _fire(table_hbm, idx0, rows0, sem0)

        _drain(table_hbm, idx1, rows1, sem1)
        _accum_store(rows1, out_v, out_hbm, row_b)
        return carry

    lax.fori_loop(0, NPAIR, pair_body, 0)


@jax.jit
def _embed(ids_t, table):
    mesh = plsc.VectorSubcoreMesh(
        core_axis_name="c",
        subcore_axis_name="s",
        num_cores=NUM_CORES,
        num_subcores=NUM_SUBCORES,
    )
    ids_flat = pl.kernel(
        _flatten_ids_body,
        out_type=jax.ShapeDtypeStruct((L * B,), jnp.int32),
        mesh=mesh,
        scratch_types=[
            pltpu.VMEM((L, B // NW), jnp.int32),
            pltpu.SemaphoreType.DMA,
        ],
        compiler_params=pltpu.CompilerParams(use_tc_tiling_on_sc=True),
    )(ids_t)
    return pl.kernel(
        _embed_body,
        out_type=jax.ShapeDtypeStruct((B, H), jnp.float32),
        mesh=mesh,
        scratch_types=[
            pltpu.VMEM((IPC,), jnp.int32),
            pltpu.VMEM((IPC,), jnp.int32),
            pltpu.VMEM((IPC, H), jnp.float32),
            pltpu.VMEM((IPC, H), jnp.float32),
            pltpu.VMEM((CB, H), jnp.float32),
            pltpu.SemaphoreType.DMA,
            pltpu.SemaphoreType.DMA,
            pltpu.SemaphoreType.DMA,
            pltpu.SemaphoreType.DMA,
        ],
        compiler_params=pltpu.CompilerParams(use_tc_tiling_on_sc=False),
    )(ids_flat, table)


def kernel(instruction_ids, embed_weight):
    ids_t = instruction_ids.astype(jnp.int32).T
    tmp = lax.optimization_barrier(jnp.swapaxes(embed_weight, 0, 1))
    return _embed(ids_t, tmp.T)


# speedup vs baseline: 2.8552x; 1.0044x over previous
import functools

import jax
import jax.numpy as jnp
from jax import lax
from jax.experimental import pallas as pl
from jax.experimental.pallas import tpu as pltpu
from jax.experimental.pallas import tpu_sc as plsc

B = 16384
L = 50
H = 32
NUM_CORES = 2
NUM_SUBCORES = 16
NW = NUM_CORES * NUM_SUBCORES
BPW = B // NW
CB = 32
NCHUNK = BPW // CB
NPAIR = NCHUNK // 2
IPC = CB * L
GSZ = 128
INV_L = 1.0 / L

_SPLITS = []
_off = 0
while _off < IPC:
    _n = min(GSZ, IPC - _off)
    _SPLITS.append((_off, _n))
    _off += _n


def _fire(table_hbm, idx_v, rows_v, sem):
    for off, n in _SPLITS:
        pltpu.async_copy(
            table_hbm.at[idx_v.at[pl.ds(off, n)]],
            rows_v.at[pl.ds(off, n)],
            sem,
        )


def _drain(table_hbm, idx_v, rows_v, sem):
    for off, n in _SPLITS:
        pltpu.make_async_copy(
            table_hbm.at[idx_v.at[pl.ds(off, n)]],
            rows_v.at[pl.ds(off, n)],
            sem,
        ).wait()


def _stage_ids(ids_hbm, col0, idx_v, sem):
    for l in range(L):
        pltpu.async_copy(
            ids_hbm.at[pl.ds(l * B + col0, CB)],
            idx_v.at[pl.ds(l * CB, CB)],
            sem,
        )
    for l in range(L):
        pltpu.make_async_copy(
            ids_hbm.at[pl.ds(l * B + col0, CB)],
            idx_v.at[pl.ds(l * CB, CB)],
            sem,
        ).wait()


def _flatten_ids_body(ids_t_hbm, out_hbm, stg, sem):
    c = lax.axis_index("c")
    s = lax.axis_index("s")
    wid = s * NUM_CORES + c
    col0 = wid * (B // NW)
    pltpu.sync_copy(ids_t_hbm.at[pl.ds(0, L), pl.ds(col0, B // NW)], stg)
    for l in range(L):
        pltpu.async_copy(
            stg.at[l, :],
            out_hbm.at[pl.ds(l * B + col0, B // NW)],
            sem,
        )
    for l in range(L):
        pltpu.make_async_copy(
            stg.at[l, :],
            out_hbm.at[pl.ds(l * B + col0, B // NW)],
            sem,
        ).wait()


def _accum_store(rows_v, out_v, out_hbm, row0):
    def row_body(b, carry):
        acc0 = jnp.zeros((16,), jnp.float32)
        acc1 = jnp.zeros((16,), jnp.float32)
        for l in range(L):
            acc0 = acc0 + rows_v[l * CB + b, pl.ds(0, 16)]
            acc1 = acc1 + rows_v[l * CB + b, pl.ds(16, 16)]
        out_v[b, pl.ds(0, 16)] = acc0 * INV_L
        out_v[b, pl.ds(16, 16)] = acc1 * INV_L
        return carry

    lax.fori_loop(0, CB, row_body, 0)
    pltpu.sync_copy(out_v, out_hbm.at[pl.ds(row0, CB)])


def _embed_body(ids_hbm, table_hbm, out_hbm,
                idx0, idx1, rows0, rows1, out_v, sem0, sem1, semi0, semi1):
    c = lax.axis_index("c")
    s = lax.axis_index("s")
    wid = s * NUM_CORES + c
    base = wid * BPW

    _stage_ids(ids_hbm, base, idx0, semi0)
    _fire(table_hbm, idx0, rows0, sem0)

    def pair_body(i, carry):
        row_a = base + (2 * i) * CB
        row_b = row_a + CB
        _stage_ids(ids_hbm, row_b, idx1, semi1)
        _fire(table_hbm, idx1, rows1, sem1)
        _drain(table_hbm, idx0, rows0, sem0)
        _accum_store(rows0, out_v, out_hbm, row_a)

        @pl.when(i < NPAIR - 1)
        def _():
            row_c = row_b + CB
            _stage_ids(ids_hbm, row_c, idx0, semi0)
            _fire(table_hbm, idx0, rows0, sem0)

        _drain(table_hbm, idx1, rows1, sem1)
        _accum_store(rows1, out_v, out_hbm, row_b)
        return carry

    lax.fori_loop(0, NPAIR, pair_body, 0)


@jax.jit
def _embed(ids_t, table):
    mesh = plsc.VectorSubcoreMesh(
        core_axis_name="c",
        subcore_axis_name="s",
        num_cores=NUM_CORES,
        num_subcores=NUM_SUBCORES,
    )
    ids_flat = pl.kernel(
        _flatten_ids_body,
        out_type=jax.ShapeDtypeStruct((L * B,), jnp.int32),
        mesh=mesh,
        scratch_types=[
            pltpu.VMEM((L, B // NW), jnp.int32),
            pltpu.SemaphoreType.DMA,
        ],
        compiler_params=pltpu.CompilerParams(use_tc_tiling_on_sc=True),
    )(ids_t)
    return pl.kernel(
        _embed_body,
        out_type=jax.ShapeDtypeStruct((B, H), jnp.float32),
        mesh=mesh,
        scratch_types=[
            pltpu.VMEM((IPC,), jnp.int32),
            pltpu.VMEM((IPC,), jnp.int32),
            pltpu.VMEM((IPC, H), jnp.float32),
            pltpu.VMEM((IPC, H), jnp.float32),
            pltpu.VMEM((CB, H), jnp.float32),
            pltpu.SemaphoreType.DMA,
            pltpu.SemaphoreType.DMA,
            pltpu.SemaphoreType.DMA,
            pltpu.SemaphoreType.DMA,
        ],
        compiler_params=pltpu.CompilerParams(use_tc_tiling_on_sc=False),
    )(ids_flat, table)


def kernel(instruction_ids, embed_weight):
    ids_t = instruction_ids.astype(jnp.int32).T
    return _embed(ids_t, embed_weight)
